# baseline (speedup 1.0000x reference)
import functools

import jax
import jax.numpy as jnp
from jax import lax
from jax.experimental import pallas as pl
from jax.experimental.pallas import tpu as pltpu
from jax.experimental.pallas import tpu_sc as plsc

B = 16384
D = 64
NC = 2
NS = 16
NW = NC * NS
BPW = B // NW
CHUNK = 128
NCHUNK = BPW // CHUNK


def _sc_gather3(user_hbm, item_hbm, uid_hbm, iid_hbm, nid_hbm,
                u_out, p_out, n_out,
                uidx, iidx, nidx, urows, prows, nrows, sem):
    wid = lax.axis_index("s") * NC + lax.axis_index("c")
    base = wid * BPW
    pltpu.sync_copy(uid_hbm.at[wid], uidx)
    pltpu.sync_copy(iid_hbm.at[wid], iidx)
    pltpu.sync_copy(nid_hbm.at[wid], nidx)
    handles = []
    for j in range(NCHUNK):
        sl = pl.ds(j * CHUNK, CHUNK)
        handles.append(pltpu.async_copy(user_hbm.at[uidx.at[j]], urows.at[sl], sem))
        handles.append(pltpu.async_copy(item_hbm.at[iidx.at[j]], prows.at[sl], sem))
        handles.append(pltpu.async_copy(item_hbm.at[nidx.at[j]], nrows.at[sl], sem))
    for h in handles:
        h.wait()
    osl = pl.ds(base, BPW)
    pltpu.sync_copy(urows, u_out.at[osl])
    pltpu.sync_copy(prows, p_out.at[osl])
    pltpu.sync_copy(nrows, n_out.at[osl])


_gather3 = functools.partial(
    pl.kernel,
    mesh=plsc.VectorSubcoreMesh(core_axis_name="c", subcore_axis_name="s"),
    out_type=[jax.ShapeDtypeStruct((B, D), jnp.float32)] * 3,
    scratch_types=[
        pltpu.VMEM((NCHUNK, CHUNK), jnp.int32),
        pltpu.VMEM((NCHUNK, CHUNK), jnp.int32),
        pltpu.VMEM((NCHUNK, CHUNK), jnp.int32),
        pltpu.VMEM((BPW, D), jnp.float32),
        pltpu.VMEM((BPW, D), jnp.float32),
        pltpu.VMEM((BPW, D), jnp.float32),
        pltpu.SemaphoreType.DMA,
    ],
    compiler_params=pltpu.CompilerParams(use_tc_tiling_on_sc=False),
)(_sc_gather3)


BLK = 2048


def _tc_linear_body(u_ref, p_ref, n_ref, wu_ref, bu_ref, wi_ref, bi_ref,
                    uo_ref, po_ref, no_ref):
    wu = wu_ref[...]
    wi = wi_ref[...]
    dn = (((1,), (1,)), ((), ()))
    uo_ref[...] = lax.dot_general(u_ref[...], wu, dn,
                                  preferred_element_type=jnp.float32) + bu_ref[...]
    po_ref[...] = lax.dot_general(p_ref[...], wi, dn,
                                  preferred_element_type=jnp.float32) + bi_ref[...]
    no_ref[...] = lax.dot_general(n_ref[...], wi, dn,
                                  preferred_element_type=jnp.float32) + bi_ref[...]


def _tc_linear(U, P, N, Wu, bu2, Wi, bi2):
    row_spec = pl.BlockSpec((BLK, D), lambda i: (i, 0))
    w_spec = pl.BlockSpec((D, D), lambda i: (0, 0))
    b_spec = pl.BlockSpec((1, D), lambda i: (0, 0))
    return pl.pallas_call(
        _tc_linear_body,
        grid=(B // BLK,),
        in_specs=[row_spec, row_spec, row_spec, w_spec, b_spec, w_spec, b_spec],
        out_specs=[row_spec, row_spec, row_spec],
        out_shape=[jax.ShapeDtypeStruct((B, D), jnp.float32)] * 3,
    )(U, P, N, Wu, bu2, Wi, bi2)


def kernel(user_emb, item_emb, Wu, bu, Wi, bi, uid, iid, nid):
    uidr = uid.astype(jnp.int32).reshape(NW, NCHUNK, CHUNK)
    iidr = iid.astype(jnp.int32).reshape(NW, NCHUNK, CHUNK)
    nidr = nid.astype(jnp.int32).reshape(NW, NCHUNK, CHUNK)
    U, P, N = _gather3(user_emb, item_emb, uidr, iidr, nidr)
    u_r, p_r, n_r = _tc_linear(U, P, N, Wu, bu.reshape(1, D), Wi, bi.reshape(1, D))
    return (u_r, p_r, n_r, P)
